# Initial kernel scaffold; baseline (speedup 1.0000x reference)
import jax
import jax.numpy as jnp
from jax.experimental import pallas as pl


def kernel(nameA, classA, nameB, classB, W_name, W_class):
    raise NotImplementedError("write your pallas kernel here")


# speedup vs baseline: 3.1040x; 3.1040x over previous
import functools

import jax
import jax.numpy as jnp
from jax import lax
from jax.experimental import pallas as pl
from jax.experimental.pallas import tpu as pltpu
from jax.experimental.pallas import tpu_sc as plsc

_BATCH = 16384
_DIM = 128


@functools.lru_cache(maxsize=None)
def _build(batch, dim):
    info = plsc.get_sparse_core_info()
    nw = info.num_cores * info.num_subcores
    b_per_w = batch // nw

    mesh = plsc.VectorSubcoreMesh(core_axis_name="c", subcore_axis_name="s")
    out = jax.ShapeDtypeStruct((batch, dim), jnp.float32)

    @functools.partial(
        pl.kernel,
        mesh=mesh,
        out_type=[out, out, out, out],
        scratch_types=[
            pltpu.VMEM((b_per_w,), jnp.int32),
            pltpu.VMEM((b_per_w, dim), jnp.float32),
            pltpu.SemaphoreType.DMA,
        ],
    )
    def four_gathers(nameA, classA, nameB, classB, w_name, w_class,
                     outA, outCA, outB, outCB, idx_v, rows_v, sem):
        wid = lax.axis_index("s") * info.num_cores + lax.axis_index("c")
        base = wid * b_per_w
        for idx_h, table_h, out_h in (
            (nameA, w_name, outA),
            (classA, w_class, outCA),
            (nameB, w_name, outB),
            (classB, w_class, outCB),
        ):
            pltpu.sync_copy(idx_h.at[pl.ds(base, b_per_w)], idx_v)
            pltpu.async_copy(table_h.at[idx_v], rows_v, sem).wait()
            pltpu.sync_copy(rows_v, out_h.at[pl.ds(base, b_per_w)])

    return four_gathers


def kernel(nameA, classA, nameB, classB, W_name, W_class):
    fn = _build(_BATCH, _DIM)
    return tuple(fn(nameA, classA, nameB, classB, W_name, W_class))


# speedup vs baseline: 3.1275x; 1.0076x over previous
import functools

import jax
import jax.numpy as jnp
from jax import lax
from jax.experimental import pallas as pl
from jax.experimental.pallas import tpu as pltpu
from jax.experimental.pallas import tpu_sc as plsc

_BATCH = 16384
_DIM = 128
_CHUNK = 256
_NBUF = 2


@functools.lru_cache(maxsize=None)
def _build(batch, dim):
    info = plsc.get_sparse_core_info()
    nw = info.num_cores * info.num_subcores
    b_per_w = batch // nw
    n_chunks = b_per_w // _CHUNK

    mesh = plsc.VectorSubcoreMesh(core_axis_name="c", subcore_axis_name="s")
    out = jax.ShapeDtypeStruct((batch, dim), jnp.float32)

    @functools.partial(
        pl.kernel,
        mesh=mesh,
        out_type=[out, out, out, out],
        scratch_types=(
            [pltpu.VMEM((b_per_w,), jnp.int32) for _ in range(4)]
            + [pltpu.VMEM((_CHUNK, dim), jnp.float32) for _ in range(_NBUF)]
            + [pltpu.SemaphoreType.DMA]
            + [pltpu.SemaphoreType.DMA for _ in range(_NBUF)]
            + [pltpu.SemaphoreType.DMA for _ in range(_NBUF)]
        ),
    )
    def four_gathers(nameA, classA, nameB, classB, w_name, w_class,
                     outA, outCA, outB, outCB,
                     idx0, idx1, idx2, idx3,
                     *rest):
        rows = rest[:_NBUF]
        sem_idx = rest[_NBUF]
        sem_g = rest[_NBUF + 1:_NBUF + 1 + _NBUF]
        sem_w = rest[_NBUF + 1 + _NBUF:]
        wid = lax.axis_index("s") * info.num_cores + lax.axis_index("c")
        base = wid * b_per_w

        idx_bufs = (idx0, idx1, idx2, idx3)
        idx_hbm = (nameA, classA, nameB, classB)
        tables = (w_name, w_class, w_name, w_class)
        outs = (outA, outCA, outB, outCB)

        idx_handles = [
            pltpu.async_copy(idx_hbm[t].at[pl.ds(base, b_per_w)],
                             idx_bufs[t], sem_idx)
            for t in range(4)
        ]
        for h in idx_handles:
            h.wait()

        items = [(t, c) for t in range(4) for c in range(n_chunks)]
        g_handles = [None] * _NBUF
        w_handles = [None] * _NBUF
        prev = None
        for i, (t, c) in enumerate(items):
            b = i % _NBUF
            if w_handles[b] is not None:
                w_handles[b].wait()
            g_handles[b] = pltpu.async_copy(
                tables[t].at[idx_bufs[t].at[pl.ds(c * _CHUNK, _CHUNK)]],
                rows[b], sem_g[b])
            if prev is not None:
                pi, pt, pc = prev
                pb = pi % _NBUF
                g_handles[pb].wait()
                w_handles[pb] = pltpu.async_copy(
                    rows[pb],
                    outs[pt].at[pl.ds(base + pc * _CHUNK, _CHUNK)],
                    sem_w[pb])
            prev = (i, t, c)
        pi, pt, pc = prev
        pb = pi % _NBUF
        g_handles[pb].wait()
        w_handles[pb] = pltpu.async_copy(
            rows[pb], outs[pt].at[pl.ds(base + pc * _CHUNK, _CHUNK)],
            sem_w[pb])
        for h in w_handles:
            if h is not None:
                h.wait()

    return four_gathers


def kernel(nameA, classA, nameB, classB, W_name, W_class):
    fn = _build(_BATCH, _DIM)
    return tuple(fn(nameA, classA, nameB, classB, W_name, W_class))


# speedup vs baseline: 4.0847x; 1.3061x over previous
import functools

import jax
import jax.numpy as jnp
from jax import lax
from jax.experimental import pallas as pl
from jax.experimental.pallas import tpu as pltpu
from jax.experimental.pallas import tpu_sc as plsc

_BATCH = 16384
_DIM = 128
_CHUNK = 256
_NBUF = 2
_CLASS_V = 1000


@functools.lru_cache(maxsize=None)
def _build(batch, dim):
    info = plsc.get_sparse_core_info()
    nw = info.num_cores * info.num_subcores
    b_per_w = batch // nw
    n_chunks = b_per_w // _CHUNK

    mesh = plsc.VectorSubcoreMesh(core_axis_name="c", subcore_axis_name="s")
    out = jax.ShapeDtypeStruct((batch, dim), jnp.float32)

    @functools.partial(
        pl.kernel,
        mesh=mesh,
        out_type=[out, out, out, out],
        scratch_types=(
            [pltpu.VMEM((b_per_w,), jnp.int32) for _ in range(4)]
            + [pltpu.VMEM((_CHUNK, dim), jnp.float32) for _ in range(_NBUF)]
            + [pltpu.VMEM_SHARED((_CLASS_V, dim), jnp.float32)]
            + [pltpu.SemaphoreType.DMA for _ in range(2 + 2 * _NBUF)]
        ),
    )
    def four_gathers(nameA, classA, nameB, classB, w_name, w_class,
                     outA, outCA, outB, outCB,
                     idx0, idx1, idx2, idx3,
                     *rest):
        rows = rest[:_NBUF]
        wc_sh = rest[_NBUF]
        sem_idx = rest[_NBUF + 1]
        sem_stage = rest[_NBUF + 2]
        sem_g = rest[_NBUF + 3:_NBUF + 3 + _NBUF]
        sem_w = rest[_NBUF + 3 + _NBUF:]
        sid = lax.axis_index("s")
        wid = sid * info.num_cores + lax.axis_index("c")
        base = wid * b_per_w

        idx_bufs = (idx0, idx1, idx2, idx3)
        idx_hbm = (nameA, classA, nameB, classB)
        outs = (outA, outCA, outB, outCB)

        idx_handles = [
            pltpu.async_copy(idx_hbm[t].at[pl.ds(base, b_per_w)],
                             idx_bufs[t], sem_idx)
            for t in range(4)
        ]

        stage_rows = _CLASS_V // 5

        @pl.when(sid < 5)
        def _stage():
            pltpu.async_copy(
                w_class.at[pl.ds(sid * stage_rows, stage_rows)],
                wc_sh.at[pl.ds(sid * stage_rows, stage_rows)],
                sem_stage).wait()

        for h in idx_handles:
            h.wait()

        name_items = [(t, c) for t in (0, 2) for c in range(n_chunks)]
        class_items = [(t, c) for t in (1, 3) for c in range(n_chunks)]

        g_handles = [None] * _NBUF
        w_handles = [None] * _NBUF
        state = {"prev": None, "i": 0}

        def run_items(items, src_for_t):
            for (t, c) in items:
                i = state["i"]
                b = i % _NBUF
                if w_handles[b] is not None:
                    w_handles[b].wait()
                g_handles[b] = pltpu.async_copy(
                    src_for_t[t].at[idx_bufs[t].at[pl.ds(c * _CHUNK, _CHUNK)]],
                    rows[b], sem_g[b])
                if state["prev"] is not None:
                    pi, pt, pc = state["prev"]
                    pb = pi % _NBUF
                    g_handles[pb].wait()
                    w_handles[pb] = pltpu.async_copy(
                        rows[pb],
                        outs[pt].at[pl.ds(base + pc * _CHUNK, _CHUNK)],
                        sem_w[pb])
                state["prev"] = (i, t, c)
                state["i"] += 1

        srcs = {0: w_name, 2: w_name, 1: wc_sh, 3: wc_sh}
        run_items(name_items, srcs)
        plsc.subcore_barrier()
        run_items(class_items, srcs)

        pi, pt, pc = state["prev"]
        pb = pi % _NBUF
        g_handles[pb].wait()
        w_handles[pb] = pltpu.async_copy(
            rows[pb], outs[pt].at[pl.ds(base + pc * _CHUNK, _CHUNK)],
            sem_w[pb])
        for h in w_handles:
            if h is not None:
                h.wait()

    return four_gathers


def kernel(nameA, classA, nameB, classB, W_name, W_class):
    fn = _build(_BATCH, _DIM)
    return tuple(fn(nameA, classA, nameB, classB, W_name, W_class))


# speedup vs baseline: 4.0874x; 1.0007x over previous
import functools

import jax
import jax.numpy as jnp
from jax import lax
from jax.experimental import pallas as pl
from jax.experimental.pallas import tpu as pltpu
from jax.experimental.pallas import tpu_sc as plsc

_BATCH = 16384
_DIM = 128
_CHUNK = 128
_NBUF = 4
_CLASS_V = 1000


@functools.lru_cache(maxsize=None)
def _build(batch, dim):
    info = plsc.get_sparse_core_info()
    nw = info.num_cores * info.num_subcores
    b_per_w = batch // nw
    n_chunks = b_per_w // _CHUNK

    mesh = plsc.VectorSubcoreMesh(core_axis_name="c", subcore_axis_name="s")
    out = jax.ShapeDtypeStruct((batch, dim), jnp.float32)

    @functools.partial(
        pl.kernel,
        mesh=mesh,
        out_type=[out, out, out, out],
        scratch_types=(
            [pltpu.VMEM((b_per_w,), jnp.int32) for _ in range(4)]
            + [pltpu.VMEM((_CHUNK, dim), jnp.float32) for _ in range(_NBUF)]
            + [pltpu.VMEM_SHARED((_CLASS_V, dim), jnp.float32)]
            + [pltpu.SemaphoreType.DMA for _ in range(2 + 2 * _NBUF)]
        ),
    )
    def four_gathers(nameA, classA, nameB, classB, w_name, w_class,
                     outA, outCA, outB, outCB,
                     idx0, idx1, idx2, idx3,
                     *rest):
        rows = rest[:_NBUF]
        wc_sh = rest[_NBUF]
        sem_idx = rest[_NBUF + 1]
        sem_stage = rest[_NBUF + 2]
        sem_g = rest[_NBUF + 3:_NBUF + 3 + _NBUF]
        sem_w = rest[_NBUF + 3 + _NBUF:]
        sid = lax.axis_index("s")
        wid = sid * info.num_cores + lax.axis_index("c")
        base = wid * b_per_w

        idx_bufs = (idx0, idx1, idx2, idx3)
        idx_hbm = (nameA, classA, nameB, classB)
        outs = (outA, outCA, outB, outCB)

        idx_handles = [
            pltpu.async_copy(idx_hbm[t].at[pl.ds(base, b_per_w)],
                             idx_bufs[t], sem_idx)
            for t in range(4)
        ]

        stage_rows = _CLASS_V // 5

        @pl.when(sid < 5)
        def _stage():
            pltpu.async_copy(
                w_class.at[pl.ds(sid * stage_rows, stage_rows)],
                wc_sh.at[pl.ds(sid * stage_rows, stage_rows)],
                sem_stage).wait()

        for h in idx_handles:
            h.wait()

        name_items = [(t, c) for t in (0, 2) for c in range(n_chunks)]
        class_items = [(t, c) for t in (1, 3) for c in range(n_chunks)]

        g_handles = [None] * _NBUF
        w_handles = [None] * _NBUF
        state = {"prev": None, "i": 0}

        def run_items(items, src_for_t):
            for (t, c) in items:
                i = state["i"]
                b = i % _NBUF
                if w_handles[b] is not None:
                    w_handles[b].wait()
                g_handles[b] = pltpu.async_copy(
                    src_for_t[t].at[idx_bufs[t].at[pl.ds(c * _CHUNK, _CHUNK)]],
                    rows[b], sem_g[b])
                if state["prev"] is not None:
                    pi, pt, pc = state["prev"]
                    pb = pi % _NBUF
                    g_handles[pb].wait()
                    w_handles[pb] = pltpu.async_copy(
                        rows[pb],
                        outs[pt].at[pl.ds(base + pc * _CHUNK, _CHUNK)],
                        sem_w[pb])
                state["prev"] = (i, t, c)
                state["i"] += 1

        srcs = {0: w_name, 2: w_name, 1: wc_sh, 3: wc_sh}
        run_items(name_items, srcs)
        plsc.subcore_barrier()
        run_items(class_items, srcs)

        pi, pt, pc = state["prev"]
        pb = pi % _NBUF
        g_handles[pb].wait()
        w_handles[pb] = pltpu.async_copy(
            rows[pb], outs[pt].at[pl.ds(base + pc * _CHUNK, _CHUNK)],
            sem_w[pb])
        for h in w_handles:
            if h is not None:
                h.wait()

    return four_gathers


def kernel(nameA, classA, nameB, classB, W_name, W_class):
    fn = _build(_BATCH, _DIM)
    return tuple(fn(nameA, classA, nameB, classB, W_name, W_class))
